# baseline (speedup 1.0000x reference)
import jax
import jax.numpy as jnp
from jax import lax
from jax.experimental import pallas as pl
from jax.experimental.pallas import tpu as pltpu, tpu_sc as plsc

N0 = 4096
K = 20
VOCAB = 1000
CHUNK = 25
BLOCKS_PER_K = VOCAB // CHUNK
N_BLOCKS = K * BLOCKS_PER_K
BLOCK_WORDS = CHUNK * N0


def _sc_onehot(xt_hbm, out_hbm, xcol_v, img_v, sem):
    info = plsc.get_sparse_core_info()
    nc = info.num_cores
    wid = lax.axis_index("s") * nc + lax.axis_index("c")
    nsub = nc * info.num_subcores
    blocks_per_sub = N_BLOCKS // nsub

    iota16 = lax.iota(jnp.int32, 16)
    zeros16 = jnp.zeros((16,), jnp.float32)
    ones16 = jnp.ones((16,), jnp.float32)

    def init_zero(c, _):
        img_v[pl.ds(c * 16, 16)] = zeros16
        return ()

    lax.fori_loop(0, BLOCK_WORDS // 16, init_zero, ())

    def run_block(t, _):
        u = wid * blocks_per_sub + t
        k = u // BLOCKS_PER_K
        v0 = (u % BLOCKS_PER_K) * CHUNK
        base = u * BLOCK_WORDS

        pltpu.sync_copy(xt_hbm.at[k], xcol_v)

        def scat(val16):
            def body(c, _):
                v16 = xcol_v[pl.ds(c * 16, 16)]
                in_blk = (v16 >= v0) & (v16 < v0 + CHUNK)
                v_cl = jnp.minimum(jnp.maximum(v16, v0), v0 + (CHUNK - 1))
                local = (v_cl - v0) * N0 + (c * 16) + iota16
                plsc.store_scatter(img_v, [local], jnp.where(in_blk, val16, zeros16))
                return ()

            lax.fori_loop(0, N0 // 16, body, ())

        scat(ones16)
        pltpu.make_async_copy(
            img_v, out_hbm.at[pl.ds(base, BLOCK_WORDS)], sem
        ).start()
        pltpu.make_async_copy(
            img_v, out_hbm.at[pl.ds(base, BLOCK_WORDS)], sem
        ).wait()
        scat(zeros16)
        return ()

    lax.fori_loop(0, blocks_per_sub, run_block, ())


def kernel(x):
    xt = x.T
    mesh = plsc.VectorSubcoreMesh(core_axis_name="c", subcore_axis_name="s")
    out_flat = pl.kernel(
        _sc_onehot,
        mesh=mesh,
        out_type=jax.ShapeDtypeStruct((K * VOCAB * N0,), jnp.float32),
        scratch_types=[
            pltpu.VMEM((N0,), jnp.int32),
            pltpu.VMEM((BLOCK_WORDS,), jnp.float32),
            pltpu.SemaphoreType.DMA,
        ],
        compiler_params=pltpu.CompilerParams(needs_layout_passes=False),
    )(xt)
    return out_flat.reshape(K, VOCAB, N0).transpose(2, 0, 1)


# speedup vs baseline: 6.1562x; 6.1562x over previous
import jax
import jax.numpy as jnp
from jax.experimental import pallas as pl
from jax.experimental.pallas import tpu as pltpu

VOCAB = 1000
BV = 40
BW = 8
NBUF = 3


def _onehot_t_ring(xt_ref, o_ref, vbuf, wsems, rsems):
    n_steps = VOCAB // BV
    n_warm = BV // BW
    k, n = xt_ref.shape
    xt = xt_ref[...]
    iota_w = jax.lax.broadcasted_iota(jnp.int32, (k, BW, n), 1)
    iota = jax.lax.broadcasted_iota(jnp.int32, (k, BV, n), 1)

    for j in range(n_warm):
        vbuf[0, :, j * BW:(j + 1) * BW, :] = (
            xt[:, None, :] == iota_w + j * BW
        ).astype(jnp.float32)
        pltpu.make_async_copy(
            vbuf.at[0, :, pl.ds(j * BW, BW), :],
            o_ref.at[:, pl.ds(j * BW, BW), :],
            wsems.at[j],
        ).start()

    def rcopy(i, slot):
        return pltpu.make_async_copy(
            vbuf.at[slot], o_ref.at[:, pl.ds(i * BV, BV), :], rsems.at[slot]
        )

    def body(i, _):
        slot = jax.lax.rem(i, NBUF)

        @pl.when(i >= NBUF + 1)
        def _():
            rcopy(i - NBUF, slot).wait()

        @pl.when(i == NBUF)
        def _():
            for j in range(n_warm):
                pltpu.make_async_copy(
                    vbuf.at[0, :, pl.ds(j * BW, BW), :],
                    o_ref.at[:, pl.ds(j * BW, BW), :],
                    wsems.at[j],
                ).wait()

        vbuf[slot] = (xt[:, None, :] == iota + i * BV).astype(jnp.float32)
        rcopy(i, slot).start()
        return ()

    jax.lax.fori_loop(1, n_steps, body, ())

    def drain(i, _):
        rcopy(i, jax.lax.rem(i, NBUF)).wait()
        return ()

    jax.lax.fori_loop(n_steps - NBUF, n_steps, drain, ())


def kernel(x):
    n0, n1 = x.shape
    xt = x.T
    out_t = pl.pallas_call(
        _onehot_t_ring,
        in_specs=[pl.BlockSpec(memory_space=pltpu.VMEM)],
        out_specs=pl.BlockSpec(memory_space=pl.ANY),
        out_shape=jax.ShapeDtypeStruct((n1, VOCAB, n0), jnp.float32),
        scratch_shapes=[
            pltpu.VMEM((NBUF, n1, BV, n0), jnp.float32),
            pltpu.SemaphoreType.DMA((BV // BW,)),
            pltpu.SemaphoreType.DMA((NBUF,)),
        ],
        compiler_params=pltpu.CompilerParams(
            vmem_limit_bytes=100 * 1024 * 1024,
        ),
    )(xt)
    return out_t.transpose(2, 0, 1)
